# baseline (speedup 1.0000x reference)
import functools

import jax
import jax.numpy as jnp
from jax import lax
from jax.experimental import pallas as pl
from jax.experimental.pallas import tpu as pltpu
from jax.experimental.pallas import tpu_sc as plsc

BATCH = 16384
NUM_FIELDS = 26
FIELD_SIZE = 100000
VOCAB = NUM_FIELDS * FIELD_SIZE

NC = 2
NS = 16
NW = NC * NS
B_PER_W = BATCH // NW
N_IDX = B_PER_W * NUM_FIELDS

RCH = 160
N_RCH = VOCAB // RCH


def _reformat_body(table_ref, flat_ref, rub0, rub1, fv0, fv1,
                   si0, si1, so0, so1):
    wid = lax.axis_index("s") * NC + lax.axis_index("c")
    cnt = 507 + (wid < N_RCH % NW).astype(jnp.int32)

    lanes = lax.iota(jnp.int32, 16)
    zeros16 = jnp.zeros((16,), jnp.int32)

    def hs(c):
        return (wid + c * NW) * RCH

    def start_in(c, rb, sem):
        pltpu.async_copy(table_ref.at[pl.ds(hs(c), RCH), :], rb, sem)

    def wait_in(c, rb, sem):
        pltpu.make_async_copy(
            table_ref.at[pl.ds(hs(c), RCH), :], rb, sem
        ).wait()

    def process(rb, fv):
        for i in range(RCH // 16):
            v = plsc.load_gather(rb, [i * 16 + lanes, zeros16])
            fv[pl.ds(i * 16, 16)] = v

    def start_out(c, fv, sem):
        pltpu.async_copy(fv, flat_ref.at[pl.ds(hs(c), RCH)], sem)

    def wait_out(c, fv, sem):
        pltpu.make_async_copy(fv, flat_ref.at[pl.ds(hs(c), RCH)], sem).wait()

    npairs = cnt // 2
    odd = cnt - 2 * npairs

    start_in(0, rub0, si0)

    def pair(i2, _):
        c0 = 2 * i2
        c1 = c0 + 1
        start_in(c1, rub1, si1)
        wait_in(c0, rub0, si0)

        @pl.when(i2 > 0)
        def _():
            wait_out(c0 - 2, fv0, so0)

        process(rub0, fv0)
        start_out(c0, fv0, so0)

        @pl.when(c0 + 2 < cnt)
        def _():
            start_in(c0 + 2, rub0, si0)

        wait_in(c1, rub1, si1)

        @pl.when(i2 > 0)
        def _():
            wait_out(c1 - 2, fv1, so1)

        process(rub1, fv1)
        start_out(c1, fv1, so1)
        return 0

    lax.fori_loop(0, npairs, pair, 0)

    last0 = 2 * npairs - 2

    @pl.when(odd == 1)
    def _():
        c = 2 * npairs
        wait_in(c, rub0, si0)
        wait_out(c - 2, fv0, so0)
        process(rub0, fv0)
        start_out(c, fv0, so0)

    wait_out(jnp.where(odd == 1, 2 * npairs, last0), fv0, so0)
    wait_out(2 * npairs - 1, fv1, so1)


def _gather_body(x_ref, table_ref, out_ref, x_v, idx_v, rows_v, out_v, sem):
    wid = lax.axis_index("s") * NC + lax.axis_index("c")
    base = wid * N_IDX

    pltpu.sync_copy(x_ref.at[pl.ds(base, N_IDX)], x_v)

    lanes26 = lax.iota(jnp.int32, 16) * NUM_FIELDS

    def build(t, _):
        f = t // (B_PER_W // 16)
        c2 = t % (B_PER_W // 16)
        xpos = lanes26 + (c2 * 16 * NUM_FIELDS + f)
        xv = plsc.load_gather(x_v, [xpos])
        idx_v[pl.ds(t * 16, 16)] = xv + f * FIELD_SIZE
        return 0

    lax.fori_loop(0, NUM_FIELDS * (B_PER_W // 16), build, 0, unroll=4)

    pltpu.async_copy(table_ref.at[idx_v], rows_v, sem).wait()

    def reduce_chunk(c2, _):
        def add_f(f, acc):
            q = f * B_PER_W + c2 * 16
            return acc + rows_v[pl.ds(q, 16)]

        acc = lax.fori_loop(
            0, NUM_FIELDS, add_f, jnp.zeros((16,), jnp.float32), unroll=4
        )
        out_v[pl.ds(c2 * 16, 16)] = acc
        return 0

    lax.fori_loop(0, B_PER_W // 16, reduce_chunk, 0)

    pltpu.sync_copy(out_v, out_ref.at[pl.ds(wid * B_PER_W, B_PER_W)])


_SC_PARAMS = pltpu.CompilerParams(
    needs_layout_passes=False, use_tc_tiling_on_sc=False
)


@jax.jit
def kernel(x, table, bias):
    mesh = plsc.VectorSubcoreMesh(core_axis_name="c", subcore_axis_name="s")
    reformat = pl.kernel(
        _reformat_body,
        out_type=jax.ShapeDtypeStruct((VOCAB,), jnp.float32),
        mesh=mesh,
        compiler_params=_SC_PARAMS,
        scratch_types=[
            pltpu.VMEM((RCH, 1), jnp.float32),
            pltpu.VMEM((RCH, 1), jnp.float32),
            pltpu.VMEM((RCH,), jnp.float32),
            pltpu.VMEM((RCH,), jnp.float32),
            pltpu.SemaphoreType.DMA,
            pltpu.SemaphoreType.DMA,
            pltpu.SemaphoreType.DMA,
            pltpu.SemaphoreType.DMA,
        ],
    )
    gather = pl.kernel(
        _gather_body,
        out_type=jax.ShapeDtypeStruct((BATCH,), jnp.float32),
        mesh=mesh,
        compiler_params=_SC_PARAMS,
        scratch_types=[
            pltpu.VMEM((N_IDX,), jnp.int32),
            pltpu.VMEM((N_IDX,), jnp.int32),
            pltpu.VMEM((N_IDX,), jnp.float32),
            pltpu.VMEM((B_PER_W,), jnp.float32),
            pltpu.SemaphoreType.DMA,
        ],
    )
    flat = reformat(table)
    out = gather(x.reshape(-1), flat)
    return out.reshape(BATCH, 1) + bias[None, :]


# speedup vs baseline: 26.1601x; 26.1601x over previous
import functools

import jax
import jax.numpy as jnp
from jax import lax
from jax.experimental import pallas as pl
from jax.experimental.pallas import tpu as pltpu
from jax.experimental.pallas import tpu_sc as plsc

BATCH = 16384
NUM_FIELDS = 26
FIELD_SIZE = 100000

NC = 2
NS = 16
NW = NC * NS
B_PER_W = BATCH // NW
N_IDX = B_PER_W * NUM_FIELDS


def _body(x_ref, table_ref, out_ref, x_v, idx_v, rows_v, out_v, sem):
    wid = lax.axis_index("s") * NC + lax.axis_index("c")
    base = wid * N_IDX

    pltpu.sync_copy(x_ref.at[pl.ds(base, N_IDX)], x_v)

    lanes26 = lax.iota(jnp.int32, 16) * NUM_FIELDS

    def build(t, _):
        f = t // (B_PER_W // 16)
        c2 = t % (B_PER_W // 16)
        xpos = lanes26 + (c2 * 16 * NUM_FIELDS + f)
        xv = plsc.load_gather(x_v, [xpos])
        idx_v[pl.ds(t * 16, 16)] = xv + f * FIELD_SIZE
        return 0

    lax.fori_loop(0, NUM_FIELDS * (B_PER_W // 16), build, 0, unroll=4)

    pltpu.async_copy(table_ref.at[idx_v], rows_v, sem).wait()

    def reduce_chunk(c2, _):
        def add_f(f, acc):
            q = f * B_PER_W + c2 * 16
            return acc + rows_v[pl.ds(q, 16)]

        acc = lax.fori_loop(
            0, NUM_FIELDS, add_f, jnp.zeros((16,), jnp.float32), unroll=4
        )
        out_v[pl.ds(c2 * 16, 16)] = acc
        return 0

    lax.fori_loop(0, B_PER_W // 16, reduce_chunk, 0)

    pltpu.sync_copy(out_v, out_ref.at[pl.ds(wid * B_PER_W, B_PER_W)])


@jax.jit
def kernel(x, table, bias):
    mesh = plsc.VectorSubcoreMesh(core_axis_name="c", subcore_axis_name="s")
    k = pl.kernel(
        _body,
        out_type=jax.ShapeDtypeStruct((BATCH,), jnp.float32),
        mesh=mesh,
        compiler_params=pltpu.CompilerParams(
            needs_layout_passes=False, use_tc_tiling_on_sc=False
        ),
        scratch_types=[
            pltpu.VMEM((N_IDX,), jnp.int32),
            pltpu.VMEM((N_IDX,), jnp.int32),
            pltpu.VMEM((N_IDX,), jnp.float32),
            pltpu.VMEM((B_PER_W,), jnp.float32),
            pltpu.SemaphoreType.DMA,
        ],
    )
    out = k(x.reshape(-1), lax.squeeze(table, (1,)))
    return out.reshape(BATCH, 1) + bias[None, :]
